# Initial kernel scaffold; baseline (speedup 1.0000x reference)
import jax
import jax.numpy as jnp
from jax.experimental import pallas as pl


def kernel(x, edge_weight, W1, b1, W2, b2, W3, b3, P1, pb1, P2, pb2, P3, pb3, edge_index, pos_edge_index, neg_edge_index):
    raise NotImplementedError("write your pallas kernel here")


# speedup vs baseline: 61.1765x; 61.1765x over previous
import functools

import jax
import jax.numpy as jnp
from jax import lax
from jax.experimental import pallas as pl
from jax.experimental.pallas import tpu as pltpu
from jax.experimental.pallas import tpu_sc as plsc

N = 10000
N_PAD = 10240
D = 128
PE = 65536

NC = 2
NS = 16
LANES = 16
CHUNK = 128
ROWS_PER_SUB = N_PAD // NS

_MESH = plsc.VectorSubcoreMesh(core_axis_name="c", subcore_axis_name="s")


def _i32(v):
    return jnp.int32(v)


def _fill(buf, value):
    w = buf.shape[1]

    @pl.loop(0, CHUNK)
    def _(r):
        for j in range(w // LANES):
            buf[r, pl.ds(j * LANES, LANES)] = jnp.full((LANES,), value, jnp.float32)


def _clear_acc(zeros_buf, acc_sh, s):
    for i in range(ROWS_PER_SUB // CHUNK):
        pltpu.sync_copy(
            zeros_buf,
            acc_sh.at[pl.ds(s * _i32(ROWS_PER_SUB) + _i32(i * CHUNK), CHUNK)],
        )


def _sc_degrees(idx_cat):
    e_pad = idx_cat.shape[0] // 2
    g_per_sub = e_pad // (NS * CHUNK)
    rps = ROWS_PER_SUB

    @functools.partial(
        pl.kernel,
        mesh=_MESH,
        out_type=jax.ShapeDtypeStruct((NC, N_PAD), jnp.float32),
        scratch_types=[
            pltpu.VMEM((CHUNK,), jnp.int32),
            pltpu.VMEM((CHUNK,), jnp.int32),
            pltpu.VMEM((CHUNK,), jnp.float32),
            pltpu.VMEM_SHARED((N_PAD,), jnp.float32),
            pltpu.SemaphoreType.DMA,
            pltpu.SemaphoreType.DMA,
        ],
    )
    def deg_kernel(idx_hbm, out_hbm, idx0, idx1, ones_v, acc_sh, sem0, sem1):
        c = lax.axis_index("c")
        s = lax.axis_index("s")
        base_e = c * _i32(e_pad) + s * _i32(g_per_sub * CHUNK)

        @pl.loop(0, CHUNK // LANES)
        def _(q):
            ones_v[pl.ds(q * _i32(LANES), LANES)] = jnp.zeros((LANES,), jnp.float32)
        for i in range(rps // CHUNK):
            pltpu.sync_copy(
                ones_v, acc_sh.at[pl.ds(s * _i32(rps) + _i32(i * CHUNK), CHUNK)])

        @pl.loop(0, CHUNK // LANES)
        def _(q):
            ones_v[pl.ds(q * _i32(LANES), LANES)] = jnp.full((LANES,), 1.0, jnp.float32)
        plsc.subcore_barrier()

        idxs = (idx0, idx1)
        sems = (sem0, sem1)

        def start_idx(g, b):
            pltpu.async_copy(
                idx_hbm.at[pl.ds(base_e + g * _i32(CHUNK), CHUNK)],
                idxs[b], sems[b])

        def wait_idx(b):
            pltpu.make_async_copy(
                idx_hbm.at[pl.ds(0, CHUNK)], idxs[b], sems[b]).wait()

        start_idx(0, 0)
        start_idx(1, 1)

        @pl.loop(0, g_per_sub, step=2)
        def _(g0):
            for b in range(2):
                g = g0 + b
                wait_idx(b)
                pltpu.sync_copy(ones_v, acc_sh.at[idxs[b]], add=True)

                @pl.when(g + 2 < g_per_sub)
                def _():
                    start_idx(g + 2, b)

        plsc.subcore_barrier()
        pltpu.sync_copy(
            acc_sh.at[pl.ds(s * _i32(rps), rps)],
            out_hbm.at[c].at[pl.ds(s * _i32(rps), rps)],
        )

    return deg_kernel(idx_cat)


def _sc_edge_pass(hs, src_flat, dst_flat, ew_flat):
    e_pad = src_flat.shape[0]
    e_per_core = e_pad // NC
    g_per_sub = e_per_core // (NS * CHUNK)

    @functools.partial(
        pl.kernel,
        mesh=_MESH,
        out_type=jax.ShapeDtypeStruct((NC, N_PAD, D), jnp.float32),
        scratch_types=[
            pltpu.VMEM((CHUNK,), jnp.int32),
            pltpu.VMEM((CHUNK,), jnp.int32),
            pltpu.VMEM((CHUNK,), jnp.int32),
            pltpu.VMEM((CHUNK,), jnp.int32),
            pltpu.VMEM((CHUNK,), jnp.float32),
            pltpu.VMEM((CHUNK,), jnp.float32),
            pltpu.VMEM((CHUNK, D), jnp.float32),
            pltpu.VMEM((CHUNK, D), jnp.float32),
            pltpu.VMEM_SHARED((N_PAD, D), jnp.float32),
            pltpu.SemaphoreType.DMA,
            pltpu.SemaphoreType.DMA,
            pltpu.SemaphoreType.DMA,
            pltpu.SemaphoreType.DMA,
        ],
    )
    def edge_kernel(hs_hbm, src_hbm, dst_hbm, ew_hbm, out_hbm,
                    si0, si1, di0, di1, ew0, ew1, rows0, rows1, acc_sh,
                    semi0, semi1, semg0, semg1):
        c = lax.axis_index("c")
        s = lax.axis_index("s")
        base_e = c * _i32(e_per_core) + s * _i32(g_per_sub * CHUNK)

        _fill(rows0, 0.0)
        _clear_acc(rows0, acc_sh, s)
        plsc.subcore_barrier()

        sis, dis, ews = (si0, si1), (di0, di1), (ew0, ew1)
        rows = (rows0, rows1)
        semi, semg = (semi0, semi1), (semg0, semg1)

        def start_idx(g, b):
            off = base_e + g * _i32(CHUNK)
            pltpu.async_copy(src_hbm.at[pl.ds(off, CHUNK)], sis[b], semi[b])
            pltpu.async_copy(dst_hbm.at[pl.ds(off, CHUNK)], dis[b], semi[b])
            pltpu.async_copy(ew_hbm.at[pl.ds(off, CHUNK)], ews[b], semi[b])

        def wait_idx(b):
            pltpu.make_async_copy(
                src_hbm.at[pl.ds(0, CHUNK)], sis[b], semi[b]).wait()
            pltpu.make_async_copy(
                dst_hbm.at[pl.ds(0, CHUNK)], dis[b], semi[b]).wait()
            pltpu.make_async_copy(
                ew_hbm.at[pl.ds(0, CHUNK)], ews[b], semi[b]).wait()

        def start_gather(b):
            pltpu.async_copy(hs_hbm.at[sis[b]], rows[b], semg[b])

        def wait_gather(b):
            pltpu.make_async_copy(
                hs_hbm.at[sis[b]], rows[b], semg[b]).wait()

        start_idx(0, 0)
        start_idx(1, 1)
        wait_idx(0)
        start_gather(0)

        @pl.loop(0, g_per_sub, step=2)
        def _(g0):
            for b in range(2):
                g = g0 + b
                wait_gather(b)

                @pl.when(g + 1 < g_per_sub)
                def _():
                    wait_idx(1 - b)
                    start_gather(1 - b)

                @pl.loop(0, CHUNK // LANES)
                def _(q):
                    w16 = ews[b][pl.ds(q * _i32(LANES), LANES)]
                    for i in range(LANES):
                        w_e = w16[i]
                        r = q * _i32(LANES) + _i32(i)
                        for j in range(D // LANES):
                            sl = pl.ds(j * LANES, LANES)
                            rows[b][r, sl] = rows[b][r, sl] * w_e

                pltpu.sync_copy(rows[b], acc_sh.at[dis[b]], add=True)

                @pl.when(g + 2 < g_per_sub)
                def _():
                    start_idx(g + 2, b)

        plsc.subcore_barrier()
        pltpu.sync_copy(
            acc_sh.at[pl.ds(s * _i32(ROWS_PER_SUB), ROWS_PER_SUB)],
            out_hbm.at[c].at[pl.ds(s * _i32(ROWS_PER_SUB), ROWS_PER_SUB)],
        )

    return edge_kernel(hs, src_flat, dst_flat, ew_flat)


def _sc_pair_products(h, ia, ib):
    b_tot = ia.shape[0]
    r_per_sub = b_tot // (NC * NS)
    g_per_sub = r_per_sub // CHUNK

    @functools.partial(
        pl.kernel,
        mesh=_MESH,
        out_type=jax.ShapeDtypeStruct((b_tot, D), jnp.float32),
        scratch_types=[
            pltpu.VMEM((CHUNK,), jnp.int32),
            pltpu.VMEM((CHUNK,), jnp.int32),
            pltpu.VMEM((CHUNK,), jnp.int32),
            pltpu.VMEM((CHUNK,), jnp.int32),
            pltpu.VMEM((CHUNK, D), jnp.float32),
            pltpu.VMEM((CHUNK, D), jnp.float32),
            pltpu.VMEM((CHUNK, D), jnp.float32),
            pltpu.VMEM((CHUNK, D), jnp.float32),
            pltpu.VMEM((CHUNK, D), jnp.float32),
            pltpu.VMEM((CHUNK, D), jnp.float32),
            pltpu.SemaphoreType.DMA,
            pltpu.SemaphoreType.DMA,
            pltpu.SemaphoreType.DMA,
            pltpu.SemaphoreType.DMA,
            pltpu.SemaphoreType.DMA,
            pltpu.SemaphoreType.DMA,
        ],
    )
    def pair_kernel(h_hbm, ia_hbm, ib_hbm, out_hbm,
                    ia0, ia1, ib0, ib1, ra0, ra1, rb0, rb1, p0, p1,
                    semi0, semi1, semg0, semg1, semw0, semw1):
        c = lax.axis_index("c")
        s = lax.axis_index("s")
        base = (c * _i32(NS) + s) * _i32(r_per_sub)

        ias, ibs = (ia0, ia1), (ib0, ib1)
        ras, rbs, ps = (ra0, ra1), (rb0, rb1), (p0, p1)
        semi, semg, semw = (semi0, semi1), (semg0, semg1), (semw0, semw1)

        def start_idx(g, b):
            off = base + g * _i32(CHUNK)
            pltpu.async_copy(ia_hbm.at[pl.ds(off, CHUNK)], ias[b], semi[b])
            pltpu.async_copy(ib_hbm.at[pl.ds(off, CHUNK)], ibs[b], semi[b])

        def wait_idx(b):
            pltpu.make_async_copy(
                ia_hbm.at[pl.ds(0, CHUNK)], ias[b], semi[b]).wait()
            pltpu.make_async_copy(
                ib_hbm.at[pl.ds(0, CHUNK)], ibs[b], semi[b]).wait()

        def start_gathers(b):
            pltpu.async_copy(h_hbm.at[ias[b]], ras[b], semg[b])
            pltpu.async_copy(h_hbm.at[ibs[b]], rbs[b], semg[b])

        def wait_gathers(b):
            pltpu.make_async_copy(h_hbm.at[ias[b]], ras[b], semg[b]).wait()
            pltpu.make_async_copy(h_hbm.at[ibs[b]], rbs[b], semg[b]).wait()

        start_idx(0, 0)
        start_idx(1, 1)
        wait_idx(0)
        start_gathers(0)

        @pl.loop(0, g_per_sub, step=2)
        def _(g0):
            for b in range(2):
                g = g0 + b
                wait_gathers(b)

                @pl.when(g + 1 < g_per_sub)
                def _():
                    wait_idx(1 - b)
                    start_gathers(1 - b)

                @pl.when(g >= 2)
                def _():
                    pltpu.make_async_copy(
                        ps[b], out_hbm.at[pl.ds(0, CHUNK)], semw[b]).wait()

                @pl.loop(0, CHUNK)
                def _(r):
                    for j in range(D // LANES):
                        sl = pl.ds(j * LANES, LANES)
                        ps[b][r, sl] = ras[b][r, sl] * rbs[b][r, sl]

                pltpu.async_copy(
                    ps[b],
                    out_hbm.at[pl.ds(base + g * _i32(CHUNK), CHUNK)], semw[b])

                @pl.when(g + 2 < g_per_sub)
                def _():
                    start_idx(g + 2, b)

        pltpu.make_async_copy(p0, out_hbm.at[pl.ds(0, CHUNK)], semw0).wait()
        pltpu.make_async_copy(p1, out_hbm.at[pl.ds(0, CHUNK)], semw1).wait()

    return pair_kernel(h, ia, ib)


_BN = 1024


def _tc_norms(deg, x, pad_cnt):
    def body(deg_ref, x_ref, norms_ref, hs_ref):
        i = pl.program_id(0)
        row_ids = lax.broadcasted_iota(jnp.int32, (_BN, 1), 0)
        corr = jnp.where((row_ids == 0) & (i == 0),
                         jnp.float32(pad_cnt), jnp.float32(0.0))
        d_out = deg_ref[0, :, 0:1] - corr
        d_in = deg_ref[1, :, 0:1] - corr
        ns_ = lax.rsqrt(jnp.maximum(d_out, 1.0))
        nd_ = lax.rsqrt(jnp.maximum(d_in, 1.0))
        norms_ref[...] = jnp.concatenate([ns_, nd_], axis=1)
        hs_ref[...] = x_ref[...] * ns_

    return pl.pallas_call(
        body,
        grid=(N_PAD // _BN,),
        in_specs=[
            pl.BlockSpec((2, _BN, 1), lambda i: (0, i, 0)),
            pl.BlockSpec((_BN, D), lambda i: (i, 0)),
        ],
        out_specs=[
            pl.BlockSpec((_BN, 2), lambda i: (i, 0)),
            pl.BlockSpec((_BN, D), lambda i: (i, 0)),
        ],
        out_shape=[
            jax.ShapeDtypeStruct((N_PAD, 2), jnp.float32),
            jax.ShapeDtypeStruct((N_PAD, D), jnp.float32),
        ],
    )(deg, x)


def _tc_layer(parts, norms, W, b, relu_and_prescale):
    def body(p_ref, n_ref, w_ref, b_ref, o_ref):
        agg = (p_ref[0] + p_ref[1]) * n_ref[:, 1:2]
        y = jnp.dot(agg, w_ref[...], preferred_element_type=jnp.float32,
                    precision=lax.Precision.HIGHEST)
        y = y + b_ref[...]
        if relu_and_prescale:
            y = jnp.maximum(y, 0.0) * n_ref[:, 0:1]
        o_ref[...] = y

    return pl.pallas_call(
        body,
        grid=(N_PAD // _BN,),
        in_specs=[
            pl.BlockSpec((2, _BN, D), lambda i: (0, i, 0)),
            pl.BlockSpec((_BN, 2), lambda i: (i, 0)),
            pl.BlockSpec((D, D), lambda i: (0, 0)),
            pl.BlockSpec((1, D), lambda i: (0, 0)),
        ],
        out_specs=pl.BlockSpec((_BN, D), lambda i: (i, 0)),
        out_shape=jax.ShapeDtypeStruct((N_PAD, D), jnp.float32),
    )(parts, norms, W, b.reshape(1, D))


def _tc_mlp(z, P1, pb1, P2, pb2, P3, pb3):
    br = 2048
    b_tot = z.shape[0]

    def body(z_ref, p1_ref, b1_ref, p2_ref, b2_ref, p3_ref, b3_ref, o_ref):
        t = jnp.dot(z_ref[...], p1_ref[...], preferred_element_type=jnp.float32,
                    precision=lax.Precision.HIGHEST)
        t = t + b1_ref[...]
        t = jnp.where(t > 0, t, 0.2 * t)
        t = jnp.dot(t, p2_ref[...], preferred_element_type=jnp.float32,
                    precision=lax.Precision.HIGHEST)
        t = t + b2_ref[...]
        t = jnp.where(t > 0, t, 0.2 * t)
        y = jnp.dot(t, p3_ref[...], preferred_element_type=jnp.float32,
                    precision=lax.Precision.HIGHEST)
        o_ref[...] = y + b3_ref[...]

    h1, h2 = P1.shape[1], P2.shape[1]
    return pl.pallas_call(
        body,
        grid=(b_tot // br,),
        in_specs=[
            pl.BlockSpec((br, D), lambda i: (i, 0)),
            pl.BlockSpec((D, h1), lambda i: (0, 0)),
            pl.BlockSpec((1, h1), lambda i: (0, 0)),
            pl.BlockSpec((h1, h2), lambda i: (0, 0)),
            pl.BlockSpec((1, h2), lambda i: (0, 0)),
            pl.BlockSpec((h2, 1), lambda i: (0, 0)),
            pl.BlockSpec((1, 1), lambda i: (0, 0)),
        ],
        out_specs=pl.BlockSpec((br, 1), lambda i: (i, 0)),
        out_shape=jax.ShapeDtypeStruct((b_tot, 1), jnp.float32),
    )(z, P1, pb1.reshape(1, h1), P2, pb2.reshape(1, h2),
      P3, pb3.reshape(1, 1))


def kernel(x, edge_weight, W1, b1, W2, b2, W3, b3,
           P1, pb1, P2, pb2, P3, pb3,
           edge_index, pos_edge_index, neg_edge_index):
    h_dt = jnp.promote_types(
        jnp.promote_types(x.dtype, W1.dtype),
        jnp.promote_types(W2.dtype, W3.dtype))
    s_dt = jnp.promote_types(
        h_dt, jnp.promote_types(P1.dtype, jnp.promote_types(P2.dtype, P3.dtype)))
    with jax.enable_x64(False):
        h_pos, h_neg, h = _kernel_impl(
            x, edge_weight, W1, b1, W2, b2, W3, b3,
            P1, pb1, P2, pb2, P3, pb3,
            edge_index, pos_edge_index, neg_edge_index)
    return (h_pos.astype(s_dt), h_neg.astype(s_dt), h.astype(h_dt))


def _kernel_impl(x, edge_weight, W1, b1, W2, b2, W3, b3,
                 P1, pb1, P2, pb2, P3, pb3,
                 edge_index, pos_edge_index, neg_edge_index):
    x = jnp.pad(x.astype(jnp.float32), ((0, N_PAD - N), (0, 0)))
    W1, b1, W2, b2, W3, b3, P1, pb1, P2, pb2, P3, pb3 = (
        a.astype(jnp.float32)
        for a in (W1, b1, W2, b2, W3, b3, P1, pb1, P2, pb2, P3, pb3))
    src = edge_index[0].astype(jnp.int32)
    dst = edge_index[1].astype(jnp.int32)
    ew = edge_weight.astype(jnp.float32)

    e_real = src.shape[0]
    gran = NC * NS * 8 * CHUNK
    e_pad = -(-e_real // gran) * gran
    pad_cnt = e_pad - e_real

    src_p = jnp.pad(src, (0, pad_cnt))
    dst_p = jnp.pad(dst, (0, pad_cnt))
    ew_p = jnp.pad(ew, (0, pad_cnt))

    deg = _sc_degrees(jnp.concatenate([src_p, dst_p]))
    norms, hs = _tc_norms(deg[:, :, None], x, pad_cnt)

    parts = _sc_edge_pass(hs, src_p, dst_p, ew_p)
    hs = _tc_layer(parts, norms, W1, b1, True)
    parts = _sc_edge_pass(hs, src_p, dst_p, ew_p)
    hs = _tc_layer(parts, norms, W2, b2, True)
    parts = _sc_edge_pass(hs, src_p, dst_p, ew_p)
    h = _tc_layer(parts, norms, W3, b3, False)

    ia = jnp.concatenate(
        [pos_edge_index[0], neg_edge_index[0]]).astype(jnp.int32)
    ib = jnp.concatenate(
        [pos_edge_index[1], neg_edge_index[1]]).astype(jnp.int32)
    z = _sc_pair_products(h, ia, ib)
    scores = _tc_mlp(z, P1, pb1, P2, pb2, P3, pb3)

    return (scores[:PE], scores[PE:], h[:N])


# speedup vs baseline: 91.6043x; 1.4974x over previous
import functools

import jax
import jax.numpy as jnp
from jax import lax
from jax.experimental import pallas as pl
from jax.experimental.pallas import tpu as pltpu
from jax.experimental.pallas import tpu_sc as plsc

N = 10000
N_PAD = 10240
D = 128
PE = 65536

NC = 2
NS = 16
LANES = 16
CHUNK = 128
ROWS_PER_SUB = N_PAD // NS

_MESH = plsc.VectorSubcoreMesh(core_axis_name="c", subcore_axis_name="s")


def _i32(v):
    return jnp.int32(v)


def _fill(buf, value):
    w = buf.shape[1]

    @pl.loop(0, CHUNK)
    def _(r):
        for j in range(w // LANES):
            buf[r, pl.ds(j * LANES, LANES)] = jnp.full((LANES,), value, jnp.float32)


def _clear_acc(zeros_buf, acc_sh, s):
    for i in range(ROWS_PER_SUB // CHUNK):
        pltpu.sync_copy(
            zeros_buf,
            acc_sh.at[pl.ds(s * _i32(ROWS_PER_SUB) + _i32(i * CHUNK), CHUNK)],
        )


def _sc_degrees(idx_cat):
    e_pad = idx_cat.shape[0] // 2
    g_per_sub = e_pad // (NS * CHUNK)
    rps = ROWS_PER_SUB

    @functools.partial(
        pl.kernel,
        mesh=_MESH,
        out_type=jax.ShapeDtypeStruct((NC, N_PAD), jnp.float32),
        scratch_types=[
            pltpu.VMEM((CHUNK,), jnp.int32),
            pltpu.VMEM((CHUNK,), jnp.int32),
            pltpu.VMEM((CHUNK,), jnp.float32),
            pltpu.VMEM_SHARED((N_PAD,), jnp.float32),
            pltpu.SemaphoreType.DMA,
            pltpu.SemaphoreType.DMA,
        ],
    )
    def deg_kernel(idx_hbm, out_hbm, idx0, idx1, ones_v, acc_sh, sem0, sem1):
        c = lax.axis_index("c")
        s = lax.axis_index("s")
        base_e = c * _i32(e_pad) + s * _i32(g_per_sub * CHUNK)

        @pl.loop(0, CHUNK // LANES)
        def _(q):
            ones_v[pl.ds(q * _i32(LANES), LANES)] = jnp.zeros((LANES,), jnp.float32)
        for i in range(rps // CHUNK):
            pltpu.sync_copy(
                ones_v, acc_sh.at[pl.ds(s * _i32(rps) + _i32(i * CHUNK), CHUNK)])

        @pl.loop(0, CHUNK // LANES)
        def _(q):
            ones_v[pl.ds(q * _i32(LANES), LANES)] = jnp.full((LANES,), 1.0, jnp.float32)
        plsc.subcore_barrier()

        idxs = (idx0, idx1)
        sems = (sem0, sem1)

        def start_idx(g, b):
            pltpu.async_copy(
                idx_hbm.at[pl.ds(base_e + g * _i32(CHUNK), CHUNK)],
                idxs[b], sems[b])

        def wait_idx(b):
            pltpu.make_async_copy(
                idx_hbm.at[pl.ds(0, CHUNK)], idxs[b], sems[b]).wait()

        start_idx(0, 0)
        start_idx(1, 1)

        @pl.loop(0, g_per_sub, step=2)
        def _(g0):
            for b in range(2):
                g = g0 + b
                wait_idx(b)
                pltpu.sync_copy(ones_v, acc_sh.at[idxs[b]], add=True)

                @pl.when(g + 2 < g_per_sub)
                def _():
                    start_idx(g + 2, b)

        plsc.subcore_barrier()
        pltpu.sync_copy(
            acc_sh.at[pl.ds(s * _i32(rps), rps)],
            out_hbm.at[c].at[pl.ds(s * _i32(rps), rps)],
        )

    return deg_kernel(idx_cat)


def _sc_edge_pass(hs, src_flat, dst_flat, ew_flat):
    e_pad = src_flat.shape[0]
    e_per_core = e_pad // NC
    g_per_sub = e_per_core // (NS * CHUNK)

    @functools.partial(
        pl.kernel,
        mesh=_MESH,
        out_type=jax.ShapeDtypeStruct((NC, N_PAD, D), jnp.float32),
        scratch_types=[
            pltpu.VMEM((CHUNK,), jnp.int32),
            pltpu.VMEM((CHUNK,), jnp.int32),
            pltpu.VMEM((CHUNK,), jnp.int32),
            pltpu.VMEM((CHUNK,), jnp.int32),
            pltpu.VMEM((CHUNK,), jnp.float32),
            pltpu.VMEM((CHUNK,), jnp.float32),
            pltpu.VMEM((CHUNK, D), jnp.float32),
            pltpu.VMEM((CHUNK, D), jnp.float32),
            pltpu.VMEM_SHARED((N_PAD, D), jnp.float32),
            pltpu.SemaphoreType.DMA,
            pltpu.SemaphoreType.DMA,
            pltpu.SemaphoreType.DMA,
            pltpu.SemaphoreType.DMA,
        ],
    )
    def edge_kernel(hs_hbm, src_hbm, dst_hbm, ew_hbm, out_hbm,
                    si0, si1, di0, di1, ew0, ew1, rows0, rows1, acc_sh,
                    semi0, semi1, semg0, semg1):
        c = lax.axis_index("c")
        s = lax.axis_index("s")
        base_e = c * _i32(e_per_core) + s * _i32(g_per_sub * CHUNK)

        _fill(rows0, 0.0)
        _clear_acc(rows0, acc_sh, s)
        plsc.subcore_barrier()

        sis, dis, ews = (si0, si1), (di0, di1), (ew0, ew1)
        rows = (rows0, rows1)
        semi, semg = (semi0, semi1), (semg0, semg1)

        def start_idx(g, b):
            off = base_e + g * _i32(CHUNK)
            pltpu.async_copy(src_hbm.at[pl.ds(off, CHUNK)], sis[b], semi[b])
            pltpu.async_copy(dst_hbm.at[pl.ds(off, CHUNK)], dis[b], semi[b])
            pltpu.async_copy(ew_hbm.at[pl.ds(off, CHUNK)], ews[b], semi[b])

        def wait_idx(b):
            pltpu.make_async_copy(
                src_hbm.at[pl.ds(0, CHUNK)], sis[b], semi[b]).wait()
            pltpu.make_async_copy(
                dst_hbm.at[pl.ds(0, CHUNK)], dis[b], semi[b]).wait()
            pltpu.make_async_copy(
                ew_hbm.at[pl.ds(0, CHUNK)], ews[b], semi[b]).wait()

        def start_gather(b):
            pltpu.async_copy(hs_hbm.at[sis[b]], rows[b], semg[b])

        def wait_gather(b):
            pltpu.make_async_copy(
                hs_hbm.at[sis[b]], rows[b], semg[b]).wait()

        start_idx(0, 0)
        start_idx(1, 1)
        wait_idx(0)
        start_gather(0)

        @pl.loop(0, g_per_sub, step=2)
        def _(g0):
            for b in range(2):
                g = g0 + b
                wait_gather(b)

                @pl.when(g + 1 < g_per_sub)
                def _():
                    wait_idx(1 - b)
                    start_gather(1 - b)

                @pl.loop(0, CHUNK // LANES)
                def _(q):
                    w16 = ews[b][pl.ds(q * _i32(LANES), LANES)]
                    for i in range(LANES):
                        w_e = w16[i]
                        r = q * _i32(LANES) + _i32(i)
                        for j in range(D // LANES):
                            sl = pl.ds(j * LANES, LANES)
                            rows[b][r, sl] = rows[b][r, sl] * w_e

                pltpu.sync_copy(rows[b], acc_sh.at[dis[b]], add=True)

                @pl.when(g + 2 < g_per_sub)
                def _():
                    start_idx(g + 2, b)

        plsc.subcore_barrier()
        pltpu.sync_copy(
            acc_sh.at[pl.ds(s * _i32(ROWS_PER_SUB), ROWS_PER_SUB)],
            out_hbm.at[c].at[pl.ds(s * _i32(ROWS_PER_SUB), ROWS_PER_SUB)],
        )

    return edge_kernel(hs, src_flat, dst_flat, ew_flat)


def _sc_pair_products(h, ia, ib):
    b_tot = ia.shape[0]
    r_per_sub = b_tot // (NC * NS)
    g_per_sub = r_per_sub // CHUNK

    @functools.partial(
        pl.kernel,
        mesh=_MESH,
        out_type=jax.ShapeDtypeStruct((b_tot, D), jnp.float32),
        scratch_types=[
            pltpu.VMEM((CHUNK,), jnp.int32),
            pltpu.VMEM((CHUNK,), jnp.int32),
            pltpu.VMEM((CHUNK,), jnp.int32),
            pltpu.VMEM((CHUNK,), jnp.int32),
            pltpu.VMEM((CHUNK, D), jnp.float32),
            pltpu.VMEM((CHUNK, D), jnp.float32),
            pltpu.VMEM((CHUNK, D), jnp.float32),
            pltpu.VMEM((CHUNK, D), jnp.float32),
            pltpu.VMEM((CHUNK, D), jnp.float32),
            pltpu.VMEM((CHUNK, D), jnp.float32),
            pltpu.SemaphoreType.DMA,
            pltpu.SemaphoreType.DMA,
            pltpu.SemaphoreType.DMA,
            pltpu.SemaphoreType.DMA,
            pltpu.SemaphoreType.DMA,
            pltpu.SemaphoreType.DMA,
        ],
    )
    def pair_kernel(h_hbm, ia_hbm, ib_hbm, out_hbm,
                    ia0, ia1, ib0, ib1, ra0, ra1, rb0, rb1, p0, p1,
                    semi0, semi1, semg0, semg1, semw0, semw1):
        c = lax.axis_index("c")
        s = lax.axis_index("s")
        base = (c * _i32(NS) + s) * _i32(r_per_sub)

        ias, ibs = (ia0, ia1), (ib0, ib1)
        ras, rbs, ps = (ra0, ra1), (rb0, rb1), (p0, p1)
        semi, semg, semw = (semi0, semi1), (semg0, semg1), (semw0, semw1)

        def start_idx(g, b):
            off = base + g * _i32(CHUNK)
            pltpu.async_copy(ia_hbm.at[pl.ds(off, CHUNK)], ias[b], semi[b])
            pltpu.async_copy(ib_hbm.at[pl.ds(off, CHUNK)], ibs[b], semi[b])

        def wait_idx(b):
            pltpu.make_async_copy(
                ia_hbm.at[pl.ds(0, CHUNK)], ias[b], semi[b]).wait()
            pltpu.make_async_copy(
                ib_hbm.at[pl.ds(0, CHUNK)], ibs[b], semi[b]).wait()

        def start_gathers(b):
            pltpu.async_copy(h_hbm.at[ias[b]], ras[b], semg[b])
            pltpu.async_copy(h_hbm.at[ibs[b]], rbs[b], semg[b])

        def wait_gathers(b):
            pltpu.make_async_copy(h_hbm.at[ias[b]], ras[b], semg[b]).wait()
            pltpu.make_async_copy(h_hbm.at[ibs[b]], rbs[b], semg[b]).wait()

        start_idx(0, 0)
        start_idx(1, 1)
        wait_idx(0)
        start_gathers(0)

        @pl.loop(0, g_per_sub, step=2)
        def _(g0):
            for b in range(2):
                g = g0 + b
                wait_gathers(b)

                @pl.when(g + 1 < g_per_sub)
                def _():
                    wait_idx(1 - b)
                    start_gathers(1 - b)

                @pl.when(g >= 2)
                def _():
                    pltpu.make_async_copy(
                        ps[b], out_hbm.at[pl.ds(0, CHUNK)], semw[b]).wait()

                @pl.loop(0, CHUNK)
                def _(r):
                    for j in range(D // LANES):
                        sl = pl.ds(j * LANES, LANES)
                        ps[b][r, sl] = ras[b][r, sl] * rbs[b][r, sl]

                pltpu.async_copy(
                    ps[b],
                    out_hbm.at[pl.ds(base + g * _i32(CHUNK), CHUNK)], semw[b])

                @pl.when(g + 2 < g_per_sub)
                def _():
                    start_idx(g + 2, b)

        pltpu.make_async_copy(p0, out_hbm.at[pl.ds(0, CHUNK)], semw0).wait()
        pltpu.make_async_copy(p1, out_hbm.at[pl.ds(0, CHUNK)], semw1).wait()

    return pair_kernel(h, ia, ib)


_BN = 1024


def _tc_norms(deg, x, pad_cnt):
    def body(deg_ref, x_ref, norms_ref, hs_ref):
        i = pl.program_id(0)
        row_ids = lax.broadcasted_iota(jnp.int32, (_BN, 1), 0)
        corr = jnp.where((row_ids == 0) & (i == 0),
                         jnp.float32(pad_cnt), jnp.float32(0.0))
        d_out = deg_ref[0, :, 0:1] - corr
        d_in = deg_ref[1, :, 0:1] - corr
        ns_ = lax.rsqrt(jnp.maximum(d_out, 1.0))
        nd_ = lax.rsqrt(jnp.maximum(d_in, 1.0))
        norms_ref[...] = jnp.concatenate([ns_, nd_], axis=1)
        hs_ref[...] = x_ref[...] * ns_

    return pl.pallas_call(
        body,
        grid=(N_PAD // _BN,),
        in_specs=[
            pl.BlockSpec((2, _BN, 1), lambda i: (0, i, 0)),
            pl.BlockSpec((_BN, D), lambda i: (i, 0)),
        ],
        out_specs=[
            pl.BlockSpec((_BN, 2), lambda i: (i, 0)),
            pl.BlockSpec((_BN, D), lambda i: (i, 0)),
        ],
        out_shape=[
            jax.ShapeDtypeStruct((N_PAD, 2), jnp.float32),
            jax.ShapeDtypeStruct((N_PAD, D), jnp.float32),
        ],
    )(deg, x)


def _tc_layer(parts, norms, W, b, relu_and_prescale):
    def body(p_ref, n_ref, w_ref, b_ref, o_ref):
        agg = (p_ref[0] + p_ref[1]) * n_ref[:, 1:2]
        y = jnp.dot(agg, w_ref[...], preferred_element_type=jnp.float32,
                    precision=lax.Precision.HIGHEST)
        y = y + b_ref[...]
        if relu_and_prescale:
            y = jnp.maximum(y, 0.0) * n_ref[:, 0:1]
        o_ref[...] = y

    return pl.pallas_call(
        body,
        grid=(N_PAD // _BN,),
        in_specs=[
            pl.BlockSpec((2, _BN, D), lambda i: (0, i, 0)),
            pl.BlockSpec((_BN, 2), lambda i: (i, 0)),
            pl.BlockSpec((D, D), lambda i: (0, 0)),
            pl.BlockSpec((1, D), lambda i: (0, 0)),
        ],
        out_specs=pl.BlockSpec((_BN, D), lambda i: (i, 0)),
        out_shape=jax.ShapeDtypeStruct((N_PAD, D), jnp.float32),
    )(parts, norms, W, b.reshape(1, D))


def _tc_mlp(z, P1, pb1, P2, pb2, P3, pb3):
    br = 2048
    b_tot = z.shape[0]

    def body(z_ref, p1_ref, b1_ref, p2_ref, b2_ref, p3_ref, b3_ref, o_ref):
        t = jnp.dot(z_ref[...], p1_ref[...], preferred_element_type=jnp.float32,
                    precision=lax.Precision.HIGHEST)
        t = t + b1_ref[...]
        t = jnp.where(t > 0, t, 0.2 * t)
        t = jnp.dot(t, p2_ref[...], preferred_element_type=jnp.float32,
                    precision=lax.Precision.HIGHEST)
        t = t + b2_ref[...]
        t = jnp.where(t > 0, t, 0.2 * t)
        y = jnp.dot(t, p3_ref[...], preferred_element_type=jnp.float32,
                    precision=lax.Precision.HIGHEST)
        o_ref[...] = y + b3_ref[...]

    h1, h2 = P1.shape[1], P2.shape[1]
    return pl.pallas_call(
        body,
        grid=(b_tot // br,),
        in_specs=[
            pl.BlockSpec((br, D), lambda i: (i, 0)),
            pl.BlockSpec((D, h1), lambda i: (0, 0)),
            pl.BlockSpec((1, h1), lambda i: (0, 0)),
            pl.BlockSpec((h1, h2), lambda i: (0, 0)),
            pl.BlockSpec((1, h2), lambda i: (0, 0)),
            pl.BlockSpec((h2, 1), lambda i: (0, 0)),
            pl.BlockSpec((1, 1), lambda i: (0, 0)),
        ],
        out_specs=pl.BlockSpec((br, 1), lambda i: (i, 0)),
        out_shape=jax.ShapeDtypeStruct((b_tot, 1), jnp.float32),
    )(z, P1, pb1.reshape(1, h1), P2, pb2.reshape(1, h2),
      P3, pb3.reshape(1, 1))


def kernel(x, edge_weight, W1, b1, W2, b2, W3, b3,
           P1, pb1, P2, pb2, P3, pb3,
           edge_index, pos_edge_index, neg_edge_index):
    h_dt = jnp.promote_types(
        jnp.promote_types(x.dtype, W1.dtype),
        jnp.promote_types(W2.dtype, W3.dtype))
    s_dt = jnp.promote_types(
        h_dt, jnp.promote_types(P1.dtype, jnp.promote_types(P2.dtype, P3.dtype)))
    with jax.enable_x64(False):
        h_pos, h_neg, h = _kernel_impl(
            x, edge_weight, W1, b1, W2, b2, W3, b3,
            P1, pb1, P2, pb2, P3, pb3,
            edge_index, pos_edge_index, neg_edge_index)
    return (h_pos.astype(s_dt), h_neg.astype(s_dt), h.astype(h_dt))


def _kernel_impl(x, edge_weight, W1, b1, W2, b2, W3, b3,
                 P1, pb1, P2, pb2, P3, pb3,
                 edge_index, pos_edge_index, neg_edge_index):
    x = jnp.pad(x.astype(jnp.float32), ((0, N_PAD - N), (0, 0)))
    W1, b1, W2, b2, W3, b3, P1, pb1, P2, pb2, P3, pb3 = (
        a.astype(jnp.float32)
        for a in (W1, b1, W2, b2, W3, b3, P1, pb1, P2, pb2, P3, pb3))
    src = edge_index[0].astype(jnp.int32)
    dst = edge_index[1].astype(jnp.int32)
    ew = edge_weight.astype(jnp.float32)

    e_real = src.shape[0]
    gran = NC * NS * 8 * CHUNK
    e_pad = -(-e_real // gran) * gran
    pad_cnt = e_pad - e_real

    pad_rows = _i32(N) + (jnp.arange(pad_cnt, dtype=jnp.int32) % _i32(N_PAD - N))
    src_p = jnp.concatenate([src, pad_rows])
    dst_p = jnp.concatenate([dst, pad_rows])
    ew_p = jnp.pad(ew, (0, pad_cnt))

    deg = _sc_degrees(jnp.concatenate([src_p, dst_p]))
    norms, hs = _tc_norms(deg[:, :, None], x, 0)

    parts = _sc_edge_pass(hs, src_p, dst_p, ew_p)
    hs = _tc_layer(parts, norms, W1, b1, True)
    parts = _sc_edge_pass(hs, src_p, dst_p, ew_p)
    hs = _tc_layer(parts, norms, W2, b2, True)
    parts = _sc_edge_pass(hs, src_p, dst_p, ew_p)
    h = _tc_layer(parts, norms, W3, b3, False)

    ia = jnp.concatenate(
        [pos_edge_index[0], neg_edge_index[0]]).astype(jnp.int32)
    ib = jnp.concatenate(
        [pos_edge_index[1], neg_edge_index[1]]).astype(jnp.int32)
    z = _sc_pair_products(h, ia, ib)
    scores = _tc_mlp(z, P1, pb1, P2, pb2, P3, pb3)

    return (scores[:PE], scores[PE:], h[:N])


# speedup vs baseline: 99.8120x; 1.0896x over previous
import functools

import jax
import jax.numpy as jnp
from jax import lax
from jax.experimental import pallas as pl
from jax.experimental.pallas import tpu as pltpu
from jax.experimental.pallas import tpu_sc as plsc

N = 10000
N_PAD = 10240
D = 128
PE = 65536

NC = 2
NS = 16
LANES = 16
CHUNK = 128
ROWS_PER_SUB = N_PAD // NS

_MESH = plsc.VectorSubcoreMesh(core_axis_name="c", subcore_axis_name="s")


def _i32(v):
    return jnp.int32(v)


def _fill(buf, value):
    w = buf.shape[1]

    @pl.loop(0, CHUNK)
    def _(r):
        for j in range(w // LANES):
            buf[r, pl.ds(j * LANES, LANES)] = jnp.full((LANES,), value, jnp.float32)


def _clear_acc(zeros_buf, acc_sh, s):
    for i in range(ROWS_PER_SUB // CHUNK):
        pltpu.sync_copy(
            zeros_buf,
            acc_sh.at[pl.ds(s * _i32(ROWS_PER_SUB) + _i32(i * CHUNK), CHUNK)],
        )


def _sc_degrees(idx_cat):
    e_pad = idx_cat.shape[0] // 2
    g_per_sub = e_pad // (NS * CHUNK)
    rps = ROWS_PER_SUB

    @functools.partial(
        pl.kernel,
        mesh=_MESH,
        out_type=jax.ShapeDtypeStruct((NC, N_PAD), jnp.float32),
        scratch_types=[
            pltpu.VMEM((CHUNK,), jnp.int32),
            pltpu.VMEM((CHUNK,), jnp.int32),
            pltpu.VMEM((CHUNK,), jnp.float32),
            pltpu.VMEM_SHARED((N_PAD,), jnp.float32),
            pltpu.SemaphoreType.DMA,
            pltpu.SemaphoreType.DMA,
        ],
    )
    def deg_kernel(idx_hbm, out_hbm, idx0, idx1, ones_v, acc_sh, sem0, sem1):
        c = lax.axis_index("c")
        s = lax.axis_index("s")
        base_e = c * _i32(e_pad) + s * _i32(g_per_sub * CHUNK)

        @pl.loop(0, CHUNK // LANES)
        def _(q):
            ones_v[pl.ds(q * _i32(LANES), LANES)] = jnp.zeros((LANES,), jnp.float32)
        for i in range(rps // CHUNK):
            pltpu.sync_copy(
                ones_v, acc_sh.at[pl.ds(s * _i32(rps) + _i32(i * CHUNK), CHUNK)])

        @pl.loop(0, CHUNK // LANES)
        def _(q):
            ones_v[pl.ds(q * _i32(LANES), LANES)] = jnp.full((LANES,), 1.0, jnp.float32)
        plsc.subcore_barrier()

        idxs = (idx0, idx1)
        sems = (sem0, sem1)

        def start_idx(g, b):
            pltpu.async_copy(
                idx_hbm.at[pl.ds(base_e + g * _i32(CHUNK), CHUNK)],
                idxs[b], sems[b])

        def wait_idx(b):
            pltpu.make_async_copy(
                idx_hbm.at[pl.ds(0, CHUNK)], idxs[b], sems[b]).wait()

        start_idx(0, 0)
        start_idx(1, 1)

        @pl.loop(0, g_per_sub, step=2)
        def _(g0):
            for b in range(2):
                g = g0 + b
                wait_idx(b)
                pltpu.sync_copy(ones_v, acc_sh.at[idxs[b]], add=True)

                @pl.when(g + 2 < g_per_sub)
                def _():
                    start_idx(g + 2, b)

        plsc.subcore_barrier()
        pltpu.sync_copy(
            acc_sh.at[pl.ds(s * _i32(rps), rps)],
            out_hbm.at[c].at[pl.ds(s * _i32(rps), rps)],
        )

    return deg_kernel(idx_cat)


def _sc_edge_pass(hs, src_flat, dst_flat, ew_flat):
    e_pad = src_flat.shape[0]
    e_per_core = e_pad // NC
    g_per_sub = e_per_core // (NS * CHUNK)

    @functools.partial(
        pl.kernel,
        mesh=_MESH,
        out_type=jax.ShapeDtypeStruct((NC, N_PAD, D), jnp.float32),
        scratch_types=[
            pltpu.VMEM((CHUNK,), jnp.int32),
            pltpu.VMEM((CHUNK,), jnp.int32),
            pltpu.VMEM((CHUNK,), jnp.int32),
            pltpu.VMEM((CHUNK,), jnp.int32),
            pltpu.VMEM((CHUNK,), jnp.float32),
            pltpu.VMEM((CHUNK,), jnp.float32),
            pltpu.VMEM((CHUNK, D), jnp.float32),
            pltpu.VMEM((CHUNK, D), jnp.float32),
            pltpu.VMEM_SHARED((N_PAD, D), jnp.float32),
            pltpu.SemaphoreType.DMA,
            pltpu.SemaphoreType.DMA,
            pltpu.SemaphoreType.DMA,
            pltpu.SemaphoreType.DMA,
        ],
    )
    def edge_kernel(hs_hbm, src_hbm, dst_hbm, ew_hbm, out_hbm,
                    si0, si1, di0, di1, ew0, ew1, rows0, rows1, acc_sh,
                    semi0, semi1, semg0, semg1):
        c = lax.axis_index("c")
        s = lax.axis_index("s")
        base_e = c * _i32(e_per_core) + s * _i32(g_per_sub * CHUNK)

        _fill(rows0, 0.0)
        _clear_acc(rows0, acc_sh, s)
        plsc.subcore_barrier()

        sis, dis, ews = (si0, si1), (di0, di1), (ew0, ew1)
        rows = (rows0, rows1)
        semi, semg = (semi0, semi1), (semg0, semg1)

        def start_idx(g, b):
            off = base_e + g * _i32(CHUNK)
            pltpu.async_copy(src_hbm.at[pl.ds(off, CHUNK)], sis[b], semi[b])
            pltpu.async_copy(dst_hbm.at[pl.ds(off, CHUNK)], dis[b], semi[b])
            pltpu.async_copy(ew_hbm.at[pl.ds(off, CHUNK)], ews[b], semi[b])

        def wait_idx(b):
            pltpu.make_async_copy(
                src_hbm.at[pl.ds(0, CHUNK)], sis[b], semi[b]).wait()
            pltpu.make_async_copy(
                dst_hbm.at[pl.ds(0, CHUNK)], dis[b], semi[b]).wait()
            pltpu.make_async_copy(
                ew_hbm.at[pl.ds(0, CHUNK)], ews[b], semi[b]).wait()

        def start_gather(b):
            pltpu.async_copy(hs_hbm.at[sis[b]], rows[b], semg[b])

        def wait_gather(b):
            pltpu.make_async_copy(
                hs_hbm.at[sis[b]], rows[b], semg[b]).wait()

        start_idx(0, 0)
        start_idx(1, 1)
        wait_idx(0)
        start_gather(0)

        @pl.loop(0, g_per_sub, step=2)
        def _(g0):
            for b in range(2):
                g = g0 + b
                wait_gather(b)

                @pl.when(g + 1 < g_per_sub)
                def _():
                    wait_idx(1 - b)
                    start_gather(1 - b)

                @pl.loop(0, CHUNK // LANES)
                def _(q):
                    w16 = ews[b][pl.ds(q * _i32(LANES), LANES)]
                    for i in range(LANES):
                        w_e = w16[i]
                        r = q * _i32(LANES) + _i32(i)
                        for j in range(D // LANES):
                            sl = pl.ds(j * LANES, LANES)
                            rows[b][r, sl] = rows[b][r, sl] * w_e

                pltpu.sync_copy(rows[b], acc_sh.at[dis[b]], add=True)

                @pl.when(g + 2 < g_per_sub)
                def _():
                    start_idx(g + 2, b)

        plsc.subcore_barrier()
        pltpu.sync_copy(
            acc_sh.at[pl.ds(s * _i32(ROWS_PER_SUB), ROWS_PER_SUB)],
            out_hbm.at[c].at[pl.ds(s * _i32(ROWS_PER_SUB), ROWS_PER_SUB)],
        )

    return edge_kernel(hs, src_flat, dst_flat, ew_flat)


def _sc_pair_products(h, ia, ib):
    b_tot = ia.shape[0]
    r_per_sub = b_tot // (NC * NS)
    g_per_sub = r_per_sub // CHUNK

    @functools.partial(
        pl.kernel,
        mesh=_MESH,
        out_type=jax.ShapeDtypeStruct((b_tot, D), jnp.float32),
        scratch_types=[
            pltpu.VMEM((CHUNK,), jnp.int32),
            pltpu.VMEM((CHUNK,), jnp.int32),
            pltpu.VMEM((CHUNK,), jnp.int32),
            pltpu.VMEM((CHUNK,), jnp.int32),
            pltpu.VMEM((CHUNK, D), jnp.float32),
            pltpu.VMEM((CHUNK, D), jnp.float32),
            pltpu.VMEM((CHUNK, D), jnp.float32),
            pltpu.VMEM((CHUNK, D), jnp.float32),
            pltpu.VMEM((CHUNK, D), jnp.float32),
            pltpu.VMEM((CHUNK, D), jnp.float32),
            pltpu.SemaphoreType.DMA,
            pltpu.SemaphoreType.DMA,
            pltpu.SemaphoreType.DMA,
            pltpu.SemaphoreType.DMA,
            pltpu.SemaphoreType.DMA,
            pltpu.SemaphoreType.DMA,
        ],
    )
    def pair_kernel(h_hbm, ia_hbm, ib_hbm, out_hbm,
                    ia0, ia1, ib0, ib1, ra0, ra1, rb0, rb1, p0, p1,
                    semi0, semi1, semg0, semg1, semw0, semw1):
        c = lax.axis_index("c")
        s = lax.axis_index("s")
        base = (c * _i32(NS) + s) * _i32(r_per_sub)

        ias, ibs = (ia0, ia1), (ib0, ib1)
        ras, rbs, ps = (ra0, ra1), (rb0, rb1), (p0, p1)
        semi, semg, semw = (semi0, semi1), (semg0, semg1), (semw0, semw1)

        def start_idx(g, b):
            off = base + g * _i32(CHUNK)
            pltpu.async_copy(ia_hbm.at[pl.ds(off, CHUNK)], ias[b], semi[b])
            pltpu.async_copy(ib_hbm.at[pl.ds(off, CHUNK)], ibs[b], semi[b])

        def wait_idx(b):
            pltpu.make_async_copy(
                ia_hbm.at[pl.ds(0, CHUNK)], ias[b], semi[b]).wait()
            pltpu.make_async_copy(
                ib_hbm.at[pl.ds(0, CHUNK)], ibs[b], semi[b]).wait()

        def start_gathers(b):
            pltpu.async_copy(h_hbm.at[ias[b]], ras[b], semg[b])
            pltpu.async_copy(h_hbm.at[ibs[b]], rbs[b], semg[b])

        def wait_gathers(b):
            pltpu.make_async_copy(h_hbm.at[ias[b]], ras[b], semg[b]).wait()
            pltpu.make_async_copy(h_hbm.at[ibs[b]], rbs[b], semg[b]).wait()

        start_idx(0, 0)
        start_idx(1, 1)
        wait_idx(0)
        start_gathers(0)

        @pl.loop(0, g_per_sub, step=2)
        def _(g0):
            for b in range(2):
                g = g0 + b
                wait_gathers(b)

                @pl.when(g + 1 < g_per_sub)
                def _():
                    wait_idx(1 - b)
                    start_gathers(1 - b)

                @pl.when(g >= 2)
                def _():
                    pltpu.make_async_copy(
                        ps[b], out_hbm.at[pl.ds(0, CHUNK)], semw[b]).wait()

                @pl.loop(0, CHUNK)
                def _(r):
                    for j in range(D // LANES):
                        sl = pl.ds(j * LANES, LANES)
                        ps[b][r, sl] = ras[b][r, sl] * rbs[b][r, sl]

                pltpu.async_copy(
                    ps[b],
                    out_hbm.at[pl.ds(base + g * _i32(CHUNK), CHUNK)], semw[b])

                @pl.when(g + 2 < g_per_sub)
                def _():
                    start_idx(g + 2, b)

        pltpu.make_async_copy(p0, out_hbm.at[pl.ds(0, CHUNK)], semw0).wait()
        pltpu.make_async_copy(p1, out_hbm.at[pl.ds(0, CHUNK)], semw1).wait()

    return pair_kernel(h, ia, ib)


_BN = 1024


def _tc_norms(deg, x, pad_cnt):
    def body(deg_ref, x_ref, norms_ref, hs_ref):
        i = pl.program_id(0)
        row_ids = lax.broadcasted_iota(jnp.int32, (_BN, 1), 0)
        corr = jnp.where((row_ids == 0) & (i == 0),
                         jnp.float32(pad_cnt), jnp.float32(0.0))
        d_out = deg_ref[0, :, 0:1] - corr
        d_in = deg_ref[1, :, 0:1] - corr
        ns_ = lax.rsqrt(jnp.maximum(d_out, 1.0))
        nd_ = lax.rsqrt(jnp.maximum(d_in, 1.0))
        norms_ref[...] = jnp.concatenate([ns_, nd_], axis=1)
        hs_ref[...] = x_ref[...] * ns_

    return pl.pallas_call(
        body,
        grid=(N_PAD // _BN,),
        in_specs=[
            pl.BlockSpec((2, _BN, 1), lambda i: (0, i, 0)),
            pl.BlockSpec((_BN, D), lambda i: (i, 0)),
        ],
        out_specs=[
            pl.BlockSpec((_BN, 2), lambda i: (i, 0)),
            pl.BlockSpec((_BN, D), lambda i: (i, 0)),
        ],
        out_shape=[
            jax.ShapeDtypeStruct((N_PAD, 2), jnp.float32),
            jax.ShapeDtypeStruct((N_PAD, D), jnp.float32),
        ],
    )(deg, x)


def _tc_layer(parts, norms, W, b, relu_and_prescale):
    def body(p_ref, n_ref, w_ref, b_ref, o_ref):
        agg = (p_ref[0] + p_ref[1]) * n_ref[:, 1:2]
        y = jnp.dot(agg, w_ref[...], preferred_element_type=jnp.float32,
                    precision=lax.Precision.HIGHEST)
        y = y + b_ref[...]
        if relu_and_prescale:
            y = jnp.maximum(y, 0.0) * n_ref[:, 0:1]
        o_ref[...] = y

    return pl.pallas_call(
        body,
        grid=(N_PAD // _BN,),
        in_specs=[
            pl.BlockSpec((2, _BN, D), lambda i: (0, i, 0)),
            pl.BlockSpec((_BN, 2), lambda i: (i, 0)),
            pl.BlockSpec((D, D), lambda i: (0, 0)),
            pl.BlockSpec((1, D), lambda i: (0, 0)),
        ],
        out_specs=pl.BlockSpec((_BN, D), lambda i: (i, 0)),
        out_shape=jax.ShapeDtypeStruct((N_PAD, D), jnp.float32),
    )(parts, norms, W, b.reshape(1, D))


def _tc_mlp(z, P1, pb1, P2, pb2, P3, pb3):
    br = 2048
    b_tot = z.shape[0]

    def body(z_ref, p1_ref, b1_ref, p2_ref, b2_ref, p3_ref, b3_ref, o_ref):
        t = jnp.dot(z_ref[...], p1_ref[...], preferred_element_type=jnp.float32)
        t = t + b1_ref[...]
        t = jnp.where(t > 0, t, 0.2 * t)
        t = jnp.dot(t, p2_ref[...], preferred_element_type=jnp.float32)
        t = t + b2_ref[...]
        t = jnp.where(t > 0, t, 0.2 * t)
        y = jnp.dot(t, p3_ref[...], preferred_element_type=jnp.float32)
        o_ref[...] = y + b3_ref[...]

    h1, h2 = P1.shape[1], P2.shape[1]
    return pl.pallas_call(
        body,
        grid=(b_tot // br,),
        in_specs=[
            pl.BlockSpec((br, D), lambda i: (i, 0)),
            pl.BlockSpec((D, h1), lambda i: (0, 0)),
            pl.BlockSpec((1, h1), lambda i: (0, 0)),
            pl.BlockSpec((h1, h2), lambda i: (0, 0)),
            pl.BlockSpec((1, h2), lambda i: (0, 0)),
            pl.BlockSpec((h2, 1), lambda i: (0, 0)),
            pl.BlockSpec((1, 1), lambda i: (0, 0)),
        ],
        out_specs=pl.BlockSpec((br, 1), lambda i: (i, 0)),
        out_shape=jax.ShapeDtypeStruct((b_tot, 1), jnp.float32),
    )(z, P1, pb1.reshape(1, h1), P2, pb2.reshape(1, h2),
      P3, pb3.reshape(1, 1))


def kernel(x, edge_weight, W1, b1, W2, b2, W3, b3,
           P1, pb1, P2, pb2, P3, pb3,
           edge_index, pos_edge_index, neg_edge_index):
    h_dt = jnp.promote_types(
        jnp.promote_types(x.dtype, W1.dtype),
        jnp.promote_types(W2.dtype, W3.dtype))
    s_dt = jnp.promote_types(
        h_dt, jnp.promote_types(P1.dtype, jnp.promote_types(P2.dtype, P3.dtype)))
    with jax.enable_x64(False):
        scores, h = _kernel_impl(
            x, edge_weight, W1, b1, W2, b2, W3, b3,
            P1, pb1, P2, pb2, P3, pb3,
            edge_index, pos_edge_index, neg_edge_index)
        scores2d = scores.reshape(-1, 128)
    s64 = scores2d.astype(s_dt)
    half = s64.shape[0] // 2
    h_pos = s64[:half].reshape(PE, 1)
    h_neg = s64[half:].reshape(PE, 1)
    return (h_pos, h_neg, h.astype(h_dt))


def _kernel_impl(x, edge_weight, W1, b1, W2, b2, W3, b3,
                 P1, pb1, P2, pb2, P3, pb3,
                 edge_index, pos_edge_index, neg_edge_index):
    x = jnp.pad(x.astype(jnp.float32), ((0, N_PAD - N), (0, 0)))
    W1, b1, W2, b2, W3, b3, P1, pb1, P2, pb2, P3, pb3 = (
        a.astype(jnp.float32)
        for a in (W1, b1, W2, b2, W3, b3, P1, pb1, P2, pb2, P3, pb3))
    src = edge_index[0].astype(jnp.int32)
    dst = edge_index[1].astype(jnp.int32)
    ew = edge_weight.astype(jnp.float32)

    e_real = src.shape[0]
    gran = NC * NS * 8 * CHUNK
    e_pad = -(-e_real // gran) * gran
    pad_cnt = e_pad - e_real

    pad_rows = _i32(N) + (jnp.arange(pad_cnt, dtype=jnp.int32) % _i32(N_PAD - N))
    src_p = jnp.concatenate([src, pad_rows])
    dst_p = jnp.concatenate([dst, pad_rows])
    ew_p = jnp.pad(ew, (0, pad_cnt))

    deg = _sc_degrees(jnp.concatenate([src_p, dst_p]))
    norms, hs = _tc_norms(deg[:, :, None], x, 0)

    parts = _sc_edge_pass(hs, src_p, dst_p, ew_p)
    hs = _tc_layer(parts, norms, W1, b1, True)
    parts = _sc_edge_pass(hs, src_p, dst_p, ew_p)
    hs = _tc_layer(parts, norms, W2, b2, True)
    parts = _sc_edge_pass(hs, src_p, dst_p, ew_p)
    h = _tc_layer(parts, norms, W3, b3, False)

    ia = jnp.concatenate(
        [pos_edge_index[0], neg_edge_index[0]]).astype(jnp.int32)
    ib = jnp.concatenate(
        [pos_edge_index[1], neg_edge_index[1]]).astype(jnp.int32)
    z = _sc_pair_products(h, ia, ib)
    scores = _tc_mlp(z, P1, pb1, P2, pb2, P3, pb3)

    return (scores, h[:N])


# speedup vs baseline: 238.8004x; 2.3925x over previous
import functools

import jax
import jax.numpy as jnp
from jax import lax
from jax.experimental import pallas as pl
from jax.experimental.pallas import tpu as pltpu
from jax.experimental.pallas import tpu_sc as plsc

N = 10000
N_PAD = 10240
D = 128
PE = 65536

NC = 2
NS = 16
LANES = 16
CHUNK = 128
ROWS_PER_SUB = N_PAD // NS

_MESH = plsc.VectorSubcoreMesh(core_axis_name="c", subcore_axis_name="s")


def _i32(v):
    return jnp.int32(v)


def _fill(buf, value):
    w = buf.shape[1]

    @pl.loop(0, CHUNK)
    def _(r):
        for j in range(w // LANES):
            buf[r, pl.ds(j * LANES, LANES)] = jnp.full((LANES,), value, jnp.float32)


def _clear_acc(zeros_buf, acc_sh, s):
    for i in range(ROWS_PER_SUB // CHUNK):
        pltpu.sync_copy(
            zeros_buf,
            acc_sh.at[pl.ds(s * _i32(ROWS_PER_SUB) + _i32(i * CHUNK), CHUNK)],
        )


def _sc_degrees(idx_cat):
    e_pad = idx_cat.shape[0] // 2
    g_per_sub = e_pad // (NS * CHUNK)
    rps = ROWS_PER_SUB

    @functools.partial(
        pl.kernel,
        mesh=_MESH,
        out_type=jax.ShapeDtypeStruct((NC, N_PAD), jnp.float32),
        scratch_types=[
            pltpu.VMEM((CHUNK,), jnp.int32),
            pltpu.VMEM((CHUNK,), jnp.int32),
            pltpu.VMEM((CHUNK,), jnp.float32),
            pltpu.VMEM_SHARED((N_PAD,), jnp.float32),
            pltpu.SemaphoreType.DMA,
            pltpu.SemaphoreType.DMA,
        ],
    )
    def deg_kernel(idx_hbm, out_hbm, idx0, idx1, ones_v, acc_sh, sem0, sem1):
        c = lax.axis_index("c")
        s = lax.axis_index("s")
        base_e = c * _i32(e_pad) + s * _i32(g_per_sub * CHUNK)

        @pl.loop(0, CHUNK // LANES)
        def _(q):
            ones_v[pl.ds(q * _i32(LANES), LANES)] = jnp.zeros((LANES,), jnp.float32)
        for i in range(rps // CHUNK):
            pltpu.sync_copy(
                ones_v, acc_sh.at[pl.ds(s * _i32(rps) + _i32(i * CHUNK), CHUNK)])

        @pl.loop(0, CHUNK // LANES)
        def _(q):
            ones_v[pl.ds(q * _i32(LANES), LANES)] = jnp.full((LANES,), 1.0, jnp.float32)
        plsc.subcore_barrier()

        idxs = (idx0, idx1)
        sems = (sem0, sem1)

        def start_idx(g, b):
            pltpu.async_copy(
                idx_hbm.at[pl.ds(base_e + g * _i32(CHUNK), CHUNK)],
                idxs[b], sems[b])

        def wait_idx(b):
            pltpu.make_async_copy(
                idx_hbm.at[pl.ds(0, CHUNK)], idxs[b], sems[b]).wait()

        start_idx(0, 0)
        start_idx(1, 1)

        @pl.loop(0, g_per_sub, step=2)
        def _(g0):
            for b in range(2):
                g = g0 + b
                wait_idx(b)
                pltpu.sync_copy(ones_v, acc_sh.at[idxs[b]], add=True)

                @pl.when(g + 2 < g_per_sub)
                def _():
                    start_idx(g + 2, b)

        plsc.subcore_barrier()
        pltpu.sync_copy(
            acc_sh.at[pl.ds(s * _i32(rps), rps)],
            out_hbm.at[c].at[pl.ds(s * _i32(rps), rps)],
        )

    return deg_kernel(idx_cat)


def _sc_edge_pass(hs, src_flat, dst_flat, ew_flat):
    e_pad = src_flat.shape[0]
    e_per_core = e_pad // NC
    g_per_sub = e_per_core // (NS * CHUNK)

    @functools.partial(
        pl.kernel,
        mesh=_MESH,
        out_type=jax.ShapeDtypeStruct((NC, N_PAD, D), jnp.float32),
        scratch_types=[
            pltpu.VMEM((CHUNK,), jnp.int32),
            pltpu.VMEM((CHUNK,), jnp.int32),
            pltpu.VMEM((CHUNK,), jnp.int32),
            pltpu.VMEM((CHUNK,), jnp.int32),
            pltpu.VMEM((CHUNK,), jnp.float32),
            pltpu.VMEM((CHUNK,), jnp.float32),
            pltpu.VMEM((CHUNK, D), jnp.float32),
            pltpu.VMEM((CHUNK, D), jnp.float32),
            pltpu.VMEM_SHARED((N_PAD, D), jnp.float32),
            pltpu.SemaphoreType.DMA,
            pltpu.SemaphoreType.DMA,
            pltpu.SemaphoreType.DMA,
            pltpu.SemaphoreType.DMA,
        ],
    )
    def edge_kernel(hs_hbm, src_hbm, dst_hbm, ew_hbm, out_hbm,
                    si0, si1, di0, di1, ew0, ew1, rows0, rows1, acc_sh,
                    semi0, semi1, semg0, semg1):
        c = lax.axis_index("c")
        s = lax.axis_index("s")
        base_e = c * _i32(e_per_core) + s * _i32(g_per_sub * CHUNK)

        _fill(rows0, 0.0)
        _clear_acc(rows0, acc_sh, s)
        plsc.subcore_barrier()

        sis, dis, ews = (si0, si1), (di0, di1), (ew0, ew1)
        rows = (rows0, rows1)
        semi, semg = (semi0, semi1), (semg0, semg1)

        def start_idx(g, b):
            off = base_e + g * _i32(CHUNK)
            pltpu.async_copy(src_hbm.at[pl.ds(off, CHUNK)], sis[b], semi[b])
            pltpu.async_copy(dst_hbm.at[pl.ds(off, CHUNK)], dis[b], semi[b])
            pltpu.async_copy(ew_hbm.at[pl.ds(off, CHUNK)], ews[b], semi[b])

        def wait_idx(b):
            pltpu.make_async_copy(
                src_hbm.at[pl.ds(0, CHUNK)], sis[b], semi[b]).wait()
            pltpu.make_async_copy(
                dst_hbm.at[pl.ds(0, CHUNK)], dis[b], semi[b]).wait()
            pltpu.make_async_copy(
                ew_hbm.at[pl.ds(0, CHUNK)], ews[b], semi[b]).wait()

        def start_gather(b):
            pltpu.async_copy(hs_hbm.at[sis[b]], rows[b], semg[b])

        def wait_gather(b):
            pltpu.make_async_copy(
                hs_hbm.at[sis[b]], rows[b], semg[b]).wait()

        start_idx(0, 0)
        start_idx(1, 1)
        wait_idx(0)
        start_gather(0)

        @pl.loop(0, g_per_sub, step=2)
        def _(g0):
            for b in range(2):
                g = g0 + b
                wait_gather(b)

                @pl.when(g + 1 < g_per_sub)
                def _():
                    wait_idx(1 - b)
                    start_gather(1 - b)

                @pl.loop(0, CHUNK // LANES)
                def _(q):
                    w16 = ews[b][pl.ds(q * _i32(LANES), LANES)]
                    for i in range(LANES):
                        w_e = w16[i]
                        r = q * _i32(LANES) + _i32(i)
                        for j in range(D // LANES):
                            sl = pl.ds(j * LANES, LANES)
                            rows[b][r, sl] = rows[b][r, sl] * w_e

                pltpu.sync_copy(rows[b], acc_sh.at[dis[b]], add=True)

                @pl.when(g + 2 < g_per_sub)
                def _():
                    start_idx(g + 2, b)

        plsc.subcore_barrier()
        pltpu.sync_copy(
            acc_sh.at[pl.ds(s * _i32(ROWS_PER_SUB), ROWS_PER_SUB)],
            out_hbm.at[c].at[pl.ds(s * _i32(ROWS_PER_SUB), ROWS_PER_SUB)],
        )

    return edge_kernel(hs, src_flat, dst_flat, ew_flat)


def _sc_pair_products(h, ia, ib):
    b_tot = ia.shape[0]
    r_per_sub = b_tot // (NC * NS)
    g_per_sub = r_per_sub // CHUNK

    @functools.partial(
        pl.kernel,
        mesh=_MESH,
        out_type=jax.ShapeDtypeStruct((b_tot, D), jnp.float32),
        scratch_types=[
            pltpu.VMEM((CHUNK,), jnp.int32),
            pltpu.VMEM((CHUNK,), jnp.int32),
            pltpu.VMEM((CHUNK,), jnp.int32),
            pltpu.VMEM((CHUNK,), jnp.int32),
            pltpu.VMEM((CHUNK, D), jnp.float32),
            pltpu.VMEM((CHUNK, D), jnp.float32),
            pltpu.VMEM((CHUNK, D), jnp.float32),
            pltpu.VMEM((CHUNK, D), jnp.float32),
            pltpu.VMEM((CHUNK, D), jnp.float32),
            pltpu.VMEM((CHUNK, D), jnp.float32),
            pltpu.SemaphoreType.DMA,
            pltpu.SemaphoreType.DMA,
            pltpu.SemaphoreType.DMA,
            pltpu.SemaphoreType.DMA,
            pltpu.SemaphoreType.DMA,
            pltpu.SemaphoreType.DMA,
        ],
    )
    def pair_kernel(h_hbm, ia_hbm, ib_hbm, out_hbm,
                    ia0, ia1, ib0, ib1, ra0, ra1, rb0, rb1, p0, p1,
                    semi0, semi1, semg0, semg1, semw0, semw1):
        c = lax.axis_index("c")
        s = lax.axis_index("s")
        base = (c * _i32(NS) + s) * _i32(r_per_sub)

        ias, ibs = (ia0, ia1), (ib0, ib1)
        ras, rbs, ps = (ra0, ra1), (rb0, rb1), (p0, p1)
        semi, semg, semw = (semi0, semi1), (semg0, semg1), (semw0, semw1)

        def start_idx(g, b):
            off = base + g * _i32(CHUNK)
            pltpu.async_copy(ia_hbm.at[pl.ds(off, CHUNK)], ias[b], semi[b])
            pltpu.async_copy(ib_hbm.at[pl.ds(off, CHUNK)], ibs[b], semi[b])

        def wait_idx(b):
            pltpu.make_async_copy(
                ia_hbm.at[pl.ds(0, CHUNK)], ias[b], semi[b]).wait()
            pltpu.make_async_copy(
                ib_hbm.at[pl.ds(0, CHUNK)], ibs[b], semi[b]).wait()

        def start_gathers(b):
            pltpu.async_copy(h_hbm.at[ias[b]], ras[b], semg[b])
            pltpu.async_copy(h_hbm.at[ibs[b]], rbs[b], semg[b])

        def wait_gathers(b):
            pltpu.make_async_copy(h_hbm.at[ias[b]], ras[b], semg[b]).wait()
            pltpu.make_async_copy(h_hbm.at[ibs[b]], rbs[b], semg[b]).wait()

        start_idx(0, 0)
        start_idx(1, 1)
        wait_idx(0)
        start_gathers(0)

        @pl.loop(0, g_per_sub, step=2)
        def _(g0):
            for b in range(2):
                g = g0 + b
                wait_gathers(b)

                @pl.when(g + 1 < g_per_sub)
                def _():
                    wait_idx(1 - b)
                    start_gathers(1 - b)

                @pl.when(g >= 2)
                def _():
                    pltpu.make_async_copy(
                        ps[b], out_hbm.at[pl.ds(0, CHUNK)], semw[b]).wait()

                @pl.loop(0, CHUNK)
                def _(r):
                    for j in range(D // LANES):
                        sl = pl.ds(j * LANES, LANES)
                        ps[b][r, sl] = ras[b][r, sl] * rbs[b][r, sl]

                pltpu.async_copy(
                    ps[b],
                    out_hbm.at[pl.ds(base + g * _i32(CHUNK), CHUNK)], semw[b])

                @pl.when(g + 2 < g_per_sub)
                def _():
                    start_idx(g + 2, b)

        pltpu.make_async_copy(p0, out_hbm.at[pl.ds(0, CHUNK)], semw0).wait()
        pltpu.make_async_copy(p1, out_hbm.at[pl.ds(0, CHUNK)], semw1).wait()

    return pair_kernel(h, ia, ib)


_BN = 1024


def _tc_norms(deg, x, pad_cnt):
    def body(deg_ref, x_ref, norms_ref, hs_ref):
        i = pl.program_id(0)
        row_ids = lax.broadcasted_iota(jnp.int32, (_BN, 1), 0)
        corr = jnp.where((row_ids == 0) & (i == 0),
                         jnp.float32(pad_cnt), jnp.float32(0.0))
        d_out = deg_ref[0, :, 0:1] - corr
        d_in = deg_ref[1, :, 0:1] - corr
        ns_ = lax.rsqrt(jnp.maximum(d_out, 1.0))
        nd_ = lax.rsqrt(jnp.maximum(d_in, 1.0))
        norms_ref[...] = jnp.concatenate([ns_, nd_], axis=1)
        hs_ref[...] = x_ref[...] * ns_

    return pl.pallas_call(
        body,
        grid=(N_PAD // _BN,),
        in_specs=[
            pl.BlockSpec((2, _BN, 1), lambda i: (0, i, 0)),
            pl.BlockSpec((_BN, D), lambda i: (i, 0)),
        ],
        out_specs=[
            pl.BlockSpec((_BN, 2), lambda i: (i, 0)),
            pl.BlockSpec((_BN, D), lambda i: (i, 0)),
        ],
        out_shape=[
            jax.ShapeDtypeStruct((N_PAD, 2), jnp.float32),
            jax.ShapeDtypeStruct((N_PAD, D), jnp.float32),
        ],
    )(deg, x)


def _tc_layer(parts, norms, W, b, relu_and_prescale):
    def body(p_ref, n_ref, w_ref, b_ref, o_ref):
        agg = (p_ref[0] + p_ref[1]) * n_ref[:, 1:2]
        y = jnp.dot(agg, w_ref[...], preferred_element_type=jnp.float32,
                    precision=lax.Precision.HIGHEST)
        y = y + b_ref[...]
        if relu_and_prescale:
            y = jnp.maximum(y, 0.0) * n_ref[:, 0:1]
        o_ref[...] = y

    return pl.pallas_call(
        body,
        grid=(N_PAD // _BN,),
        in_specs=[
            pl.BlockSpec((2, _BN, D), lambda i: (0, i, 0)),
            pl.BlockSpec((_BN, 2), lambda i: (i, 0)),
            pl.BlockSpec((D, D), lambda i: (0, 0)),
            pl.BlockSpec((1, D), lambda i: (0, 0)),
        ],
        out_specs=pl.BlockSpec((_BN, D), lambda i: (i, 0)),
        out_shape=jax.ShapeDtypeStruct((N_PAD, D), jnp.float32),
    )(parts, norms, W, b.reshape(1, D))


def _tc_mlp(z, P1, pb1, P2, pb2, P3, pb3):
    br = 2048
    b_tot = z.shape[0]

    def body(z_ref, p1_ref, b1_ref, p2_ref, b2_ref, p3_ref, b3_ref, o_ref):
        t = jnp.dot(z_ref[...], p1_ref[...], preferred_element_type=jnp.float32)
        t = t + b1_ref[...]
        t = jnp.where(t > 0, t, 0.2 * t)
        t = jnp.dot(t, p2_ref[...], preferred_element_type=jnp.float32)
        t = t + b2_ref[...]
        t = jnp.where(t > 0, t, 0.2 * t)
        y = jnp.dot(t, p3_ref[...], preferred_element_type=jnp.float32)
        o_ref[...] = y + b3_ref[...]

    h1, h2 = P1.shape[1], P2.shape[1]
    return pl.pallas_call(
        body,
        grid=(b_tot // br,),
        in_specs=[
            pl.BlockSpec((br, D), lambda i: (i, 0)),
            pl.BlockSpec((D, h1), lambda i: (0, 0)),
            pl.BlockSpec((1, h1), lambda i: (0, 0)),
            pl.BlockSpec((h1, h2), lambda i: (0, 0)),
            pl.BlockSpec((1, h2), lambda i: (0, 0)),
            pl.BlockSpec((h2, 1), lambda i: (0, 0)),
            pl.BlockSpec((1, 1), lambda i: (0, 0)),
        ],
        out_specs=pl.BlockSpec((br, 1), lambda i: (i, 0)),
        out_shape=jax.ShapeDtypeStruct((b_tot, 1), jnp.float32),
    )(z, P1, pb1.reshape(1, h1), P2, pb2.reshape(1, h2),
      P3, pb3.reshape(1, 1))


def kernel(x, edge_weight, W1, b1, W2, b2, W3, b3,
           P1, pb1, P2, pb2, P3, pb3,
           edge_index, pos_edge_index, neg_edge_index):
    h_dt = jnp.promote_types(
        jnp.promote_types(x.dtype, W1.dtype),
        jnp.promote_types(W2.dtype, W3.dtype))
    s_dt = jnp.promote_types(
        h_dt, jnp.promote_types(P1.dtype, jnp.promote_types(P2.dtype, P3.dtype)))
    with jax.enable_x64(False):
        scores, h = _kernel_impl(
            x, edge_weight, W1, b1, W2, b2, W3, b3,
            P1, pb1, P2, pb2, P3, pb3,
            edge_index, pos_edge_index, neg_edge_index)
        scores2d = scores.reshape(-1, 128)
    s64 = lax.optimization_barrier(scores2d.astype(s_dt))
    half = s64.shape[0] // 2
    h_pos = s64[:half].reshape(PE, 1)
    h_neg = s64[half:].reshape(PE, 1)
    return (h_pos, h_neg, h.astype(h_dt))


def _kernel_impl(x, edge_weight, W1, b1, W2, b2, W3, b3,
                 P1, pb1, P2, pb2, P3, pb3,
                 edge_index, pos_edge_index, neg_edge_index):
    x = jnp.pad(x.astype(jnp.float32), ((0, N_PAD - N), (0, 0)))
    W1, b1, W2, b2, W3, b3, P1, pb1, P2, pb2, P3, pb3 = (
        a.astype(jnp.float32)
        for a in (W1, b1, W2, b2, W3, b3, P1, pb1, P2, pb2, P3, pb3))
    src = edge_index[0].astype(jnp.int32)
    dst = edge_index[1].astype(jnp.int32)
    ew = edge_weight.astype(jnp.float32)

    e_real = src.shape[0]
    gran = NC * NS * 8 * CHUNK
    e_pad = -(-e_real // gran) * gran
    pad_cnt = e_pad - e_real

    pad_rows = _i32(N) + (jnp.arange(pad_cnt, dtype=jnp.int32) % _i32(N_PAD - N))
    src_p = jnp.concatenate([src, pad_rows])
    dst_p = jnp.concatenate([dst, pad_rows])
    ew_p = jnp.pad(ew, (0, pad_cnt))

    deg = _sc_degrees(jnp.concatenate([src_p, dst_p]))
    norms, hs = _tc_norms(deg[:, :, None], x, 0)

    parts = _sc_edge_pass(hs, src_p, dst_p, ew_p)
    hs = _tc_layer(parts, norms, W1, b1, True)
    parts = _sc_edge_pass(hs, src_p, dst_p, ew_p)
    hs = _tc_layer(parts, norms, W2, b2, True)
    parts = _sc_edge_pass(hs, src_p, dst_p, ew_p)
    h = _tc_layer(parts, norms, W3, b3, False)

    ia = jnp.concatenate(
        [pos_edge_index[0], neg_edge_index[0]]).astype(jnp.int32)
    ib = jnp.concatenate(
        [pos_edge_index[1], neg_edge_index[1]]).astype(jnp.int32)
    z = _sc_pair_products(h, ia, ib)
    scores = _tc_mlp(z, P1, pb1, P2, pb2, P3, pb3)

    return (scores, h[:N])
